# Initial kernel scaffold; baseline (speedup 1.0000x reference)
import jax
import jax.numpy as jnp
from jax.experimental import pallas as pl


def kernel(x, emb):
    raise NotImplementedError("write your pallas kernel here")


# speedup vs baseline: 1.4946x; 1.4946x over previous
import jax
import jax.numpy as jnp
from jax.experimental import pallas as pl


def _add_body(x_ref, emb_ref, o_ref):
    o_ref[...] = x_ref[...] + emb_ref[...][None]


def kernel(x, emb):
    B, T, D = x.shape
    BT = 512
    nb = T // BT
    return pl.pallas_call(
        _add_body,
        grid=(nb, B),
        in_specs=[
            pl.BlockSpec((1, BT, D), lambda i, b: (b, i, 0)),
            pl.BlockSpec((BT, D), lambda i, b: (i, 0)),
        ],
        out_specs=pl.BlockSpec((1, BT, D), lambda i, b: (b, i, 0)),
        out_shape=jax.ShapeDtypeStruct(x.shape, x.dtype),
    )(x, emb)
